# Initial kernel scaffold; baseline (speedup 1.0000x reference)
import jax
import jax.numpy as jnp
from jax.experimental import pallas as pl


def kernel(h, coord, edge_index, coord_diff, coord_cross, edge_attr, W1, b1, W2, b2, W3):
    raise NotImplementedError("write your pallas kernel here")


# speedup vs baseline: 3.7630x; 3.7630x over previous
import functools

import jax
import jax.numpy as jnp
from jax import lax
from jax.experimental import pallas as pl
from jax.experimental.pallas import tpu as pltpu
from jax.experimental.pallas import tpu_sc as plsc

F32 = jnp.float32

NC = 2
NS = 16
NW = NC * NS
L = 16

HIDDEN = 128
CH = 80


def _pre_body(h_ref, wa_ref, wb_ref, b1_ref, a_ref, b_ref):
    hb = h_ref[...]
    a_ref[...] = (jnp.dot(hb, wa_ref[...], preferred_element_type=F32)
                  + b1_ref[...])
    b_ref[...] = jnp.dot(hb, wb_ref[...], preferred_element_type=F32)


def _precompute(h, W1a, W1b, b1):
    n = h.shape[0]
    bn = 1000
    return pl.pallas_call(
        _pre_body,
        grid=(n // bn,),
        in_specs=[
            pl.BlockSpec((bn, HIDDEN), lambda i: (i, 0)),
            pl.BlockSpec((HIDDEN, HIDDEN), lambda i: (0, 0)),
            pl.BlockSpec((HIDDEN, HIDDEN), lambda i: (0, 0)),
            pl.BlockSpec((1, HIDDEN), lambda i: (0, 0)),
        ],
        out_specs=[
            pl.BlockSpec((bn, HIDDEN), lambda i: (i, 0)),
            pl.BlockSpec((bn, HIDDEN), lambda i: (i, 0)),
        ],
        out_shape=[jax.ShapeDtypeStruct((n, HIDDEN), F32)] * 2,
    )(h, W1a, W1b, b1)


def _make_gather(n_edges):
    ew = n_edges // NW
    nch = ew // CH
    mesh = plsc.VectorSubcoreMesh(core_axis_name="c", subcore_axis_name="s")

    @functools.partial(
        pl.kernel,
        out_type=[jax.ShapeDtypeStruct((n_edges, HIDDEN), F32)] * 2,
        mesh=mesh,
        scratch_types=[
            pltpu.VMEM((ew,), jnp.int32),
            pltpu.VMEM((ew,), jnp.int32),
            pltpu.VMEM((CH, HIDDEN), F32),
            pltpu.VMEM((CH, HIDDEN), F32),
        ],
    )
    def gather(a_hbm, b_hbm, row_hbm, col_hbm, g1_hbm, g2_hbm,
               rowv, colv, buf1, buf2):
        wid = lax.axis_index("s") * NC + lax.axis_index("c")
        base = wid * ew
        pltpu.sync_copy(row_hbm.at[pl.ds(base, ew)], rowv)
        pltpu.sync_copy(col_hbm.at[pl.ds(base, ew)], colv)

        def body(j, carry):
            off = j * CH
            pltpu.sync_copy(a_hbm.at[rowv.at[pl.ds(off, CH)]], buf1)
            pltpu.sync_copy(b_hbm.at[colv.at[pl.ds(off, CH)]], buf2)
            pltpu.sync_copy(buf1, g1_hbm.at[pl.ds(base + off, CH)])
            pltpu.sync_copy(buf2, g2_hbm.at[pl.ds(base + off, CH)])
            return carry

        lax.fori_loop(0, nch, body, 0)

    return gather


def _mlp_body(g1_ref, g2_ref, attr_ref, cd_ref, w1c_ref, b2_ref, w2_ref,
              w3_ref, out_ref):
    pre = g1_ref[...] + g2_ref[...] + attr_ref[...] * w1c_ref[...]
    x1 = jax.nn.silu(pre)
    y = jnp.dot(x1, w2_ref[...], preferred_element_type=F32) + b2_ref[...]
    x2 = jax.nn.silu(y)
    w = jnp.dot(x2, w3_ref[...], preferred_element_type=F32)
    out_ref[...] = jnp.transpose(cd_ref[...] * w)


def _mlp(g1, g2, edge_attr, coord_diff, w1c, b2, W2, W3):
    e = g1.shape[0]
    be = 2560
    return pl.pallas_call(
        _mlp_body,
        grid=(e // be,),
        in_specs=[
            pl.BlockSpec((be, HIDDEN), lambda i: (i, 0)),
            pl.BlockSpec((be, HIDDEN), lambda i: (i, 0)),
            pl.BlockSpec((be, 1), lambda i: (i, 0)),
            pl.BlockSpec((be, 3), lambda i: (i, 0)),
            pl.BlockSpec((1, HIDDEN), lambda i: (0, 0)),
            pl.BlockSpec((1, HIDDEN), lambda i: (0, 0)),
            pl.BlockSpec((HIDDEN, HIDDEN), lambda i: (0, 0)),
            pl.BlockSpec((HIDDEN, 1), lambda i: (0, 0)),
        ],
        out_specs=pl.BlockSpec((3, be), lambda i: (0, i)),
        out_shape=jax.ShapeDtypeStruct((3, e), F32),
    )(g1, g2, edge_attr, coord_diff, w1c, b2, W2, W3)


def _make_scatter(n_edges, n_nodes):
    ew = n_edges // NW
    ngr = ew // L
    nps = ((n_nodes + NS - 1) // NS + 127) // 128 * 128
    npad = nps * NS
    mesh = plsc.VectorSubcoreMesh(core_axis_name="c", subcore_axis_name="s")

    @functools.partial(
        pl.kernel,
        out_type=[jax.ShapeDtypeStruct((npad,), F32)] * 6,
        mesh=mesh,
        scratch_types=[
            pltpu.VMEM((ew,), jnp.int32),
            pltpu.VMEM((ew,), F32),
            pltpu.VMEM((ew,), F32),
            pltpu.VMEM((ew,), F32),
            pltpu.VMEM((npad,), F32),
            pltpu.VMEM((npad,), F32),
            pltpu.VMEM((npad,), F32),
            pltpu.VMEM_SHARED((NS, npad), F32),
            pltpu.VMEM_SHARED((NS, npad), F32),
            pltpu.VMEM_SHARED((NS, npad), F32),
            pltpu.VMEM((nps,), F32),
            pltpu.VMEM((nps,), F32),
            pltpu.VMEM((nps,), F32),
            pltpu.VMEM((nps,), F32),
        ],
        compiler_params=pltpu.CompilerParams(needs_layout_passes=False),
    )
    def scatter(tx_hbm, ty_hbm, tz_hbm, row_hbm,
                px0, py0, pz0, px1, py1, pz1,
                ridx, rbx, rby, rbz, aggx, aggy, aggz,
                shx, shy, shz, accx, accy, accz, tmp):
        cid = lax.axis_index("c")
        sid = lax.axis_index("s")
        wid = sid * NC + cid
        base = wid * ew

        zero = jnp.zeros((L,), F32)

        def zrow(i, c):
            o = i * L
            aggx[pl.ds(o, L)] = zero
            aggy[pl.ds(o, L)] = zero
            aggz[pl.ds(o, L)] = zero
            return c

        lax.fori_loop(0, npad // L, zrow, 0)

        pltpu.sync_copy(row_hbm.at[pl.ds(base, ew)], ridx)
        pltpu.sync_copy(tx_hbm.at[pl.ds(base, ew)], rbx)
        pltpu.sync_copy(ty_hbm.at[pl.ds(base, ew)], rby)
        pltpu.sync_copy(tz_hbm.at[pl.ds(base, ew)], rbz)

        def body(g, carry):
            o = g * L
            idx16 = ridx[pl.ds(o, L)]
            plsc.addupdate_scatter(aggx, [idx16], rbx[pl.ds(o, L)])
            plsc.addupdate_scatter(aggy, [idx16], rby[pl.ds(o, L)])
            plsc.addupdate_scatter(aggz, [idx16], rbz[pl.ds(o, L)])
            return carry

        lax.fori_loop(0, ngr, body, 0)

        pltpu.sync_copy(aggx, shx.at[sid])
        pltpu.sync_copy(aggy, shy.at[sid])
        pltpu.sync_copy(aggz, shz.at[sid])
        plsc.subcore_barrier()

        stripe = sid * nps

        def reduce_one(sh, acc):
            pltpu.sync_copy(sh.at[0].at[pl.ds(stripe, nps)], acc)
            for t in range(1, NS):
                pltpu.sync_copy(sh.at[t].at[pl.ds(stripe, nps)], tmp)

                def addv(i, c):
                    o = i * L
                    acc[pl.ds(o, L)] = acc[pl.ds(o, L)] + tmp[pl.ds(o, L)]
                    return c

                lax.fori_loop(0, nps // L, addv, 0)

        reduce_one(shx, accx)
        reduce_one(shy, accy)
        reduce_one(shz, accz)

        @pl.when(cid == 0)
        def _():
            pltpu.sync_copy(accx, px0.at[pl.ds(stripe, nps)])
            pltpu.sync_copy(accy, py0.at[pl.ds(stripe, nps)])
            pltpu.sync_copy(accz, pz0.at[pl.ds(stripe, nps)])

        @pl.when(cid == 1)
        def _():
            pltpu.sync_copy(accx, px1.at[pl.ds(stripe, nps)])
            pltpu.sync_copy(accy, py1.at[pl.ds(stripe, nps)])
            pltpu.sync_copy(accz, pz1.at[pl.ds(stripe, nps)])

    return scatter


def _combine_body(coord_ref, px0_ref, py0_ref, pz0_ref,
                  px1_ref, py1_ref, pz1_ref, out_ref):
    n = coord_ref.shape[0]
    ax = px0_ref[pl.ds(0, n)] + px1_ref[pl.ds(0, n)]
    ay = py0_ref[pl.ds(0, n)] + py1_ref[pl.ds(0, n)]
    az = pz0_ref[pl.ds(0, n)] + pz1_ref[pl.ds(0, n)]
    agg = jnp.transpose(jnp.stack([ax, ay, az]))
    out_ref[...] = coord_ref[...] + agg * F32(0.01)


def _combine(coord, parts):
    n = coord.shape[0]
    npad = parts[0].shape[0]
    return pl.pallas_call(
        _combine_body,
        grid=(1,),
        in_specs=[pl.BlockSpec((n, 3), lambda i: (0, 0))]
        + [pl.BlockSpec((npad,), lambda i: (0,))] * 6,
        out_specs=pl.BlockSpec((n, 3), lambda i: (0, 0)),
        out_shape=jax.ShapeDtypeStruct((n, 3), F32),
    )(coord, *parts)


def kernel(h, coord, edge_index, coord_diff, coord_cross, edge_attr,
           W1, b1, W2, b2, W3):
    del coord_cross
    row = edge_index[0].astype(jnp.int32)
    col = edge_index[1].astype(jnp.int32)
    W1a = W1[:HIDDEN]
    W1b = W1[HIDDEN:2 * HIDDEN]
    w1c = W1[2 * HIDDEN:2 * HIDDEN + 1]
    b1r = b1.reshape(1, HIDDEN)
    b2r = b2.reshape(1, HIDDEN)

    a_tab, b_tab = _precompute(h, W1a, W1b, b1r)

    e = row.shape[0]
    g1, g2 = _make_gather(e)(a_tab, b_tab, row, col)
    t_t = _mlp(g1, g2, edge_attr, coord_diff, w1c, b2r, W2, W3)
    parts = _make_scatter(e, coord.shape[0])(t_t[0], t_t[1], t_t[2], row)
    return _combine(coord, parts)


# speedup vs baseline: 4.3291x; 1.1504x over previous
import functools

import jax
import jax.numpy as jnp
from jax import lax
from jax.experimental import pallas as pl
from jax.experimental.pallas import tpu as pltpu
from jax.experimental.pallas import tpu_sc as plsc

F32 = jnp.float32

NC = 2
NS = 16
NW = NC * NS
L = 16

HIDDEN = 128
CH = 40
NUN = 5
SUP = CH * NUN


def _pre_body(h_ref, wa_ref, wb_ref, b1_ref, a_ref, b_ref):
    hb = h_ref[...]
    a_ref[...] = (jnp.dot(hb, wa_ref[...], preferred_element_type=F32)
                  + b1_ref[...])
    b_ref[...] = jnp.dot(hb, wb_ref[...], preferred_element_type=F32)


def _precompute(h, W1a, W1b, b1):
    n = h.shape[0]
    bn = 1000
    return pl.pallas_call(
        _pre_body,
        grid=(n // bn,),
        in_specs=[
            pl.BlockSpec((bn, HIDDEN), lambda i: (i, 0)),
            pl.BlockSpec((HIDDEN, HIDDEN), lambda i: (0, 0)),
            pl.BlockSpec((HIDDEN, HIDDEN), lambda i: (0, 0)),
            pl.BlockSpec((1, HIDDEN), lambda i: (0, 0)),
        ],
        out_specs=[
            pl.BlockSpec((bn, HIDDEN), lambda i: (i, 0)),
            pl.BlockSpec((bn, HIDDEN), lambda i: (i, 0)),
        ],
        out_shape=[jax.ShapeDtypeStruct((n, HIDDEN), F32)] * 2,
    )(h, W1a, W1b, b1)


def _make_gather(n_edges):
    ew = n_edges // NW
    nit = ew // (2 * SUP)
    mesh = plsc.VectorSubcoreMesh(core_axis_name="c", subcore_axis_name="s")

    @functools.partial(
        pl.kernel,
        out_type=[jax.ShapeDtypeStruct((n_edges, HIDDEN), F32)] * 2,
        mesh=mesh,
        scratch_types=[
            pltpu.VMEM((ew,), jnp.int32),
            pltpu.VMEM((ew,), jnp.int32),
            pltpu.VMEM((SUP, HIDDEN), F32),
            pltpu.VMEM((SUP, HIDDEN), F32),
            pltpu.VMEM((SUP, HIDDEN), F32),
            pltpu.VMEM((SUP, HIDDEN), F32),
            pltpu.SemaphoreType.DMA,
            pltpu.SemaphoreType.DMA,
            pltpu.SemaphoreType.DMA,
            pltpu.SemaphoreType.DMA,
        ],
    )
    def gather(a_hbm, b_hbm, row_hbm, col_hbm, g1_hbm, g2_hbm,
               rowv, colv, a1, a2, b1, b2, semga, semgb, semwa, semwb):
        wid = lax.axis_index("s") * NC + lax.axis_index("c")
        base = wid * ew
        pltpu.sync_copy(row_hbm.at[pl.ds(base, ew)], rowv)
        pltpu.sync_copy(col_hbm.at[pl.ds(base, ew)], colv)

        def fire_gathers(soff, d1, d2, sem):
            hs = []
            for u in range(NUN):
                o = soff + u * CH
                hs.append(pltpu.async_copy(
                    a_hbm.at[rowv.at[pl.ds(o, CH)]],
                    d1.at[pl.ds(u * CH, CH)], sem))
                hs.append(pltpu.async_copy(
                    b_hbm.at[colv.at[pl.ds(o, CH)]],
                    d2.at[pl.ds(u * CH, CH)], sem))
            return hs

        def fire_writes(soff, s1, s2, sem):
            pltpu.async_copy(s1, g1_hbm.at[pl.ds(base + soff, SUP)], sem)
            pltpu.async_copy(s2, g2_hbm.at[pl.ds(base + soff, SUP)], sem)

        def drain_writes(s1, sem):
            for _ in range(2):
                pltpu.make_async_copy(
                    s1, g1_hbm.at[pl.ds(base, SUP)], sem).wait()

        pltpu.async_copy(a_hbm.at[pl.ds(0, SUP)], b1, semwb)
        pltpu.async_copy(a_hbm.at[pl.ds(0, SUP)], b2, semwb)

        def body(k, carry):
            s0 = 2 * k * SUP
            s1 = s0 + SUP
            ga = fire_gathers(s0, a1, a2, semga)
            drain_writes(b1, semwb)
            gb = fire_gathers(s1, b1, b2, semgb)
            for h in ga:
                h.wait()
            fire_writes(s0, a1, a2, semwa)
            for h in gb:
                h.wait()
            fire_writes(s1, b1, b2, semwb)
            drain_writes(a1, semwa)
            return carry

        lax.fori_loop(0, nit, body, 0)
        drain_writes(b1, semwb)

    return gather


def _mlp_body(g1_ref, g2_ref, attr_ref, cd_ref, w1c_ref, b2_ref, w2_ref,
              w3_ref, out_ref):
    pre = g1_ref[...] + g2_ref[...] + attr_ref[...] * w1c_ref[...]
    x1 = jax.nn.silu(pre)
    y = jnp.dot(x1, w2_ref[...], preferred_element_type=F32) + b2_ref[...]
    x2 = jax.nn.silu(y)
    w = jnp.dot(x2, w3_ref[...], preferred_element_type=F32)
    out_ref[...] = jnp.transpose(cd_ref[...] * w)


def _mlp(g1, g2, edge_attr, coord_diff, w1c, b2, W2, W3):
    e = g1.shape[0]
    be = 2560
    return pl.pallas_call(
        _mlp_body,
        grid=(e // be,),
        in_specs=[
            pl.BlockSpec((be, HIDDEN), lambda i: (i, 0)),
            pl.BlockSpec((be, HIDDEN), lambda i: (i, 0)),
            pl.BlockSpec((be, 1), lambda i: (i, 0)),
            pl.BlockSpec((be, 3), lambda i: (i, 0)),
            pl.BlockSpec((1, HIDDEN), lambda i: (0, 0)),
            pl.BlockSpec((1, HIDDEN), lambda i: (0, 0)),
            pl.BlockSpec((HIDDEN, HIDDEN), lambda i: (0, 0)),
            pl.BlockSpec((HIDDEN, 1), lambda i: (0, 0)),
        ],
        out_specs=pl.BlockSpec((3, be), lambda i: (0, i)),
        out_shape=jax.ShapeDtypeStruct((3, e), F32),
    )(g1, g2, edge_attr, coord_diff, w1c, b2, W2, W3)


def _make_scatter(n_edges, n_nodes):
    ew = n_edges // NW
    ngr = ew // L
    nps = ((n_nodes + NS - 1) // NS + 127) // 128 * 128
    npad = nps * NS
    mesh = plsc.VectorSubcoreMesh(core_axis_name="c", subcore_axis_name="s")

    @functools.partial(
        pl.kernel,
        out_type=[jax.ShapeDtypeStruct((npad,), F32)] * 6,
        mesh=mesh,
        scratch_types=[
            pltpu.VMEM((ew,), jnp.int32),
            pltpu.VMEM((ew,), F32),
            pltpu.VMEM((ew,), F32),
            pltpu.VMEM((ew,), F32),
            pltpu.VMEM((npad,), F32),
            pltpu.VMEM((npad,), F32),
            pltpu.VMEM((npad,), F32),
            pltpu.VMEM_SHARED((NS, npad), F32),
            pltpu.VMEM_SHARED((NS, npad), F32),
            pltpu.VMEM_SHARED((NS, npad), F32),
            pltpu.VMEM((nps,), F32),
            pltpu.VMEM((nps,), F32),
            pltpu.VMEM((nps,), F32),
            pltpu.VMEM((nps,), F32),
        ],
        compiler_params=pltpu.CompilerParams(needs_layout_passes=False),
    )
    def scatter(tx_hbm, ty_hbm, tz_hbm, row_hbm,
                px0, py0, pz0, px1, py1, pz1,
                ridx, rbx, rby, rbz, aggx, aggy, aggz,
                shx, shy, shz, accx, accy, accz, tmp):
        cid = lax.axis_index("c")
        sid = lax.axis_index("s")
        wid = sid * NC + cid
        base = wid * ew

        zero = jnp.zeros((L,), F32)

        def zrow(i, c):
            o = i * L
            aggx[pl.ds(o, L)] = zero
            aggy[pl.ds(o, L)] = zero
            aggz[pl.ds(o, L)] = zero
            return c

        lax.fori_loop(0, npad // L, zrow, 0)

        pltpu.sync_copy(row_hbm.at[pl.ds(base, ew)], ridx)
        pltpu.sync_copy(tx_hbm.at[pl.ds(base, ew)], rbx)
        pltpu.sync_copy(ty_hbm.at[pl.ds(base, ew)], rby)
        pltpu.sync_copy(tz_hbm.at[pl.ds(base, ew)], rbz)

        def body(g, carry):
            o = g * L
            idx16 = ridx[pl.ds(o, L)]
            plsc.addupdate_scatter(aggx, [idx16], rbx[pl.ds(o, L)])
            plsc.addupdate_scatter(aggy, [idx16], rby[pl.ds(o, L)])
            plsc.addupdate_scatter(aggz, [idx16], rbz[pl.ds(o, L)])
            return carry

        lax.fori_loop(0, ngr, body, 0)

        pltpu.sync_copy(aggx, shx.at[sid])
        pltpu.sync_copy(aggy, shy.at[sid])
        pltpu.sync_copy(aggz, shz.at[sid])
        plsc.subcore_barrier()

        stripe = sid * nps

        def reduce_one(sh, acc):
            pltpu.sync_copy(sh.at[0].at[pl.ds(stripe, nps)], acc)
            for t in range(1, NS):
                pltpu.sync_copy(sh.at[t].at[pl.ds(stripe, nps)], tmp)

                def addv(i, c):
                    o = i * L
                    acc[pl.ds(o, L)] = acc[pl.ds(o, L)] + tmp[pl.ds(o, L)]
                    return c

                lax.fori_loop(0, nps // L, addv, 0)

        reduce_one(shx, accx)
        reduce_one(shy, accy)
        reduce_one(shz, accz)

        @pl.when(cid == 0)
        def _():
            pltpu.sync_copy(accx, px0.at[pl.ds(stripe, nps)])
            pltpu.sync_copy(accy, py0.at[pl.ds(stripe, nps)])
            pltpu.sync_copy(accz, pz0.at[pl.ds(stripe, nps)])

        @pl.when(cid == 1)
        def _():
            pltpu.sync_copy(accx, px1.at[pl.ds(stripe, nps)])
            pltpu.sync_copy(accy, py1.at[pl.ds(stripe, nps)])
            pltpu.sync_copy(accz, pz1.at[pl.ds(stripe, nps)])

    return scatter


def _combine_body(coord_ref, px0_ref, py0_ref, pz0_ref,
                  px1_ref, py1_ref, pz1_ref, out_ref):
    n = coord_ref.shape[0]
    ax = px0_ref[pl.ds(0, n)] + px1_ref[pl.ds(0, n)]
    ay = py0_ref[pl.ds(0, n)] + py1_ref[pl.ds(0, n)]
    az = pz0_ref[pl.ds(0, n)] + pz1_ref[pl.ds(0, n)]
    agg = jnp.transpose(jnp.stack([ax, ay, az]))
    out_ref[...] = coord_ref[...] + agg * F32(0.01)


def _combine(coord, parts):
    n = coord.shape[0]
    npad = parts[0].shape[0]
    return pl.pallas_call(
        _combine_body,
        grid=(1,),
        in_specs=[pl.BlockSpec((n, 3), lambda i: (0, 0))]
        + [pl.BlockSpec((npad,), lambda i: (0,))] * 6,
        out_specs=pl.BlockSpec((n, 3), lambda i: (0, 0)),
        out_shape=jax.ShapeDtypeStruct((n, 3), F32),
    )(coord, *parts)


def kernel(h, coord, edge_index, coord_diff, coord_cross, edge_attr,
           W1, b1, W2, b2, W3):
    del coord_cross
    row = edge_index[0].astype(jnp.int32)
    col = edge_index[1].astype(jnp.int32)
    W1a = W1[:HIDDEN]
    W1b = W1[HIDDEN:2 * HIDDEN]
    w1c = W1[2 * HIDDEN:2 * HIDDEN + 1]
    b1r = b1.reshape(1, HIDDEN)
    b2r = b2.reshape(1, HIDDEN)

    a_tab, b_tab = _precompute(h, W1a, W1b, b1r)

    e = row.shape[0]
    g1, g2 = _make_gather(e)(a_tab, b_tab, row, col)
    t_t = _mlp(g1, g2, edge_attr, coord_diff, w1c, b2r, W2, W3)
    parts = _make_scatter(e, coord.shape[0])(t_t[0], t_t[1], t_t[2], row)
    return _combine(coord, parts)
